# Initial kernel scaffold; baseline (speedup 1.0000x reference)
import jax
import jax.numpy as jnp
from jax.experimental import pallas as pl


def kernel(x, edge_index, edge_weight, Wp, bp, g0_Wl, g0_bl, g0_Wr, g0_br, g0_We, g0_att, g0_b, g1_Wl, g1_bl, g1_Wr, g1_br, g1_We, g1_att, g1_b, Wq, bq, Wk, bk, Wv, bv, Wo, bo, f_W1, f_b1, f_W2, f_b2, r_W1, r_b1, r_W2, r_b2):
    raise NotImplementedError("write your pallas kernel here")


# speedup vs baseline: 17.1257x; 17.1257x over previous
import functools
import jax
import jax.numpy as jnp
from jax import lax
from jax.experimental import pallas as pl
from jax.experimental.pallas import tpu as pltpu
from jax.experimental.pallas import tpu_sc as plsc

NC = 2
NS = 16
L = 16
CH = 80


def _vgather(x, idx):
    dn = lax.GatherDimensionNumbers(
        offset_dims=(), collapsed_slice_dims=(0,), start_index_map=(0,))
    return lax.gather(x, idx[:, None], dn, (1,),
                      mode=lax.GatherScatterMode.PROMISE_IN_BOUNDS)


def _make_edge_kernel(n, e, hd2):
    nw = NC * NS
    assert e % (nw * CH) == 0
    ew_per = e // nw
    nch = ew_per // CH
    nvec = hd2 // L
    hv = 2
    nheads = nvec // hv
    wrows = 80
    assert n % wrows == 0
    nblk = n // wrows
    nblk_ceil = (nblk + NS - 1) // NS

    mesh = plsc.VectorSubcoreMesh(core_axis_name="c", subcore_axis_name="s")

    @functools.partial(
        pl.kernel,
        out_type=(jax.ShapeDtypeStruct((2 * n, hd2), jnp.float32),
                  jax.ShapeDtypeStruct((2 * n, L), jnp.float32)),
        mesh=mesh,
        compiler_params=pltpu.CompilerParams(use_tc_tiling_on_sc=False),
        scratch_types=dict(
            acc_sh=pltpu.VMEM_SHARED((n, hd2), jnp.float32),
            accd_sh=pltpu.VMEM_SHARED((n, L), jnp.float32),
            idxs=pltpu.VMEM((CH,), jnp.int32),
            idxd=pltpu.VMEM((CH,), jnp.int32),
            ewv=pltpu.VMEM((CH,), jnp.float32),
            xlr=pltpu.VMEM((CH, hd2), jnp.float32),
            xrr=pltpu.VMEM((CH, hd2), jnp.float32),
            stage=pltpu.VMEM((CH, hd2), jnp.float32),
            staged=pltpu.VMEM((CH, L), jnp.float32),
            wei=pltpu.VMEM((hd2,), jnp.float32),
            atti=pltpu.VMEM((hd2,), jnp.float32),
            sem1=pltpu.SemaphoreType.DMA,
            sem2=pltpu.SemaphoreType.DMA,
        ),
    )
    def edge_kernel(xl_hbm, xr_hbm, src_hbm, dst_hbm, ew_hbm, we_hbm,
                    att_hbm, out_hbm, outd_hbm, acc_sh, accd_sh, idxs, idxd,
                    ewv, xlr, xrr, stage, staged, wei, atti,
                    sem1, sem2):
        cid = lax.axis_index("c")
        sid = lax.axis_index("s")
        wid = sid * NC + cid

        pltpu.sync_copy(we_hbm, wei)
        pltpu.sync_copy(att_hbm, atti)

        zv = jnp.zeros((L,), jnp.float32)

        def zrow(r, _):
            for j in range(hd2 // L):
                stage[r, pl.ds(j * L, L)] = zv
            staged[r, pl.ds(0, L)] = zv
            return 0

        lax.fori_loop(0, CH, zrow, 0)

        def zblk(b, _):
            blk = b * NS + sid

            @pl.when(blk < nblk)
            def _():
                pltpu.sync_copy(stage, acc_sh.at[pl.ds(blk * wrows, wrows)])
                pltpu.sync_copy(staged, accd_sh.at[pl.ds(blk * wrows, wrows)])

            return 0

        lax.fori_loop(0, nblk_ceil, zblk, 0)
        plsc.subcore_barrier()

        lanes = lax.iota(jnp.int32, L)
        perms = [lanes ^ s for s in (8, 4, 2, 1)]
        base0 = wid * ew_per

        def chunk(c, _):
            base = base0 + c * CH
            pltpu.sync_copy(src_hbm.at[pl.ds(base, CH)], idxs)
            pltpu.sync_copy(dst_hbm.at[pl.ds(base, CH)], idxd)
            pltpu.sync_copy(ew_hbm.at[pl.ds(base, CH)], ewv)
            cp1 = pltpu.async_copy(xl_hbm.at[idxs], xlr, sem1)
            cp2 = pltpu.async_copy(xr_hbm.at[idxd], xrr, sem2)
            cp1.wait()
            cp2.wait()

            def egroup(g, _):
                ewg = ewv[pl.ds(g * L, L)]
                for u in range(L):
                    t = g * L + u
                    w = ewg[u]
                    den = jnp.zeros((L,), jnp.float32)
                    for h in range(nheads):
                        q = jnp.zeros((L,), jnp.float32)
                        xls = []
                        for k in range(hv):
                            i = h * hv + k
                            xv = xlr[t, pl.ds(i * L, L)]
                            z = (xv + xrr[t, pl.ds(i * L, L)]
                                 + w * wei[pl.ds(i * L, L)])
                            m = jnp.where(z >= 0.0, z, 0.2 * z)
                            q = q + m * atti[pl.ds(i * L, L)]
                            xls.append(xv)
                        for p in perms:
                            q = q + _vgather(q, p)
                        ex = jnp.exp(q)
                        for k in range(hv):
                            i = h * hv + k
                            stage[t, pl.ds(i * L, L)] = xls[k] * ex
                        den = jnp.where(lanes == h, ex, den)
                    staged[t, pl.ds(0, L)] = den
                return 0

            lax.fori_loop(0, CH // L, egroup, 0)
            pltpu.sync_copy(stage, acc_sh.at[idxd], add=True)
            pltpu.sync_copy(staged, accd_sh.at[idxd], add=True)
            return 0

        lax.fori_loop(0, nch, chunk, 0)
        plsc.subcore_barrier()

        def wblk(b, _):
            blk = b * NS + sid

            @pl.when(blk < nblk)
            def _():
                r = blk * wrows
                pltpu.sync_copy(acc_sh.at[pl.ds(r, wrows)],
                                out_hbm.at[pl.ds(cid * n + r, wrows)])
                pltpu.sync_copy(accd_sh.at[pl.ds(r, wrows)],
                                outd_hbm.at[pl.ds(cid * n + r, wrows)])

            return 0

        lax.fori_loop(0, nblk_ceil, wblk, 0)

    return edge_kernel


def _gat_layer(xp, src, dst, ew, Wl, bl, Wr, br, We, att, b):
    n = xp.shape[0]
    e = src.shape[0]
    heads, c = att.shape
    hd2 = heads * c
    xl = xp @ Wl + bl
    xr = xp @ Wr + br
    wev = We.reshape(hd2)
    attv = att.reshape(hd2)
    ek = _make_edge_kernel(n, e, hd2)
    acc, accd = ek(xl, xr, src, dst, ew, wev, attv)
    den = (accd[:n] + accd[n:])[:, :heads]
    num = (acc[:n] + acc[n:]).reshape(n, heads, c)
    out = (num / (den[:, :, None] + 1e-16)).mean(axis=1) + b
    return jax.nn.relu(out)


def _temporal(xs, Wq, bq, Wk, bk, Wv, bv, Wo, bo, heads):
    tn, bn, dm = xs.shape
    hd = Wq.shape[1] // heads
    scale = hd ** -0.5
    q = (xs @ Wq + bq).reshape(tn, bn, heads, hd).transpose(1, 2, 0, 3)
    k = (xs @ Wk + bk).reshape(tn, bn, heads, hd).transpose(1, 2, 0, 3)
    v = (xs @ Wv + bv).reshape(tn, bn, heads, hd).transpose(1, 2, 0, 3)
    s = jnp.einsum('bhtd,bhsd->bhts', q, k) * scale
    w = jax.nn.softmax(s, axis=-1)
    ctx = jnp.einsum('bhts,bhsd->bhtd', w, v)
    ctx = ctx.transpose(2, 0, 1, 3).reshape(tn, bn, heads * hd)
    return ctx @ Wo + bo


def kernel(x, edge_index, edge_weight, Wp, bp, g0_Wl, g0_bl, g0_Wr, g0_br,
           g0_We, g0_att, g0_b, g1_Wl, g1_bl, g1_Wr, g1_br, g1_We, g1_att,
           g1_b, Wq, bq, Wk, bk, Wv, bv, Wo, bo, f_W1, f_b1, f_W2, f_b2,
           r_W1, r_b1, r_W2, r_b2):
    heads = g0_att.shape[0]
    src = edge_index[0]
    dst = edge_index[1]
    x_seq = x @ Wp + bp
    x_pool = x_seq.mean(axis=1)
    x_pool = _gat_layer(x_pool, src, dst, edge_weight,
                        g0_Wl, g0_bl, g0_Wr, g0_br, g0_We, g0_att, g0_b)
    x_pool = _gat_layer(x_pool, src, dst, edge_weight,
                        g1_Wl, g1_bl, g1_Wr, g1_br, g1_We, g1_att, g1_b)
    x_seq = x_seq + x_pool[:, None, :]
    node_reps = jnp.transpose(x_seq, (1, 0, 2))
    temporal_out = _temporal(node_reps, Wq, bq, Wk, bk, Wv, bv, Wo, bo, heads)
    last = temporal_out[-1]
    forecast = jax.nn.relu(jax.nn.relu(last @ f_W1 + f_b1) @ f_W2 + f_b2)
    risk = jax.nn.sigmoid(jax.nn.relu(last @ r_W1 + r_b1) @ r_W2 + r_b2)
    return (forecast, risk, temporal_out)


# speedup vs baseline: 18.0527x; 1.0541x over previous
import functools
import jax
import jax.numpy as jnp
from jax import lax
from jax.experimental import pallas as pl
from jax.experimental.pallas import tpu as pltpu
from jax.experimental.pallas import tpu_sc as plsc

NC = 2
NS = 16
L = 16
CH = 80


def _vgather(x, idx):
    dn = lax.GatherDimensionNumbers(
        offset_dims=(), collapsed_slice_dims=(0,), start_index_map=(0,))
    return lax.gather(x, idx[:, None], dn, (1,),
                      mode=lax.GatherScatterMode.PROMISE_IN_BOUNDS)


def _make_edge_kernel(n, e, hd2):
    nw = NC * NS
    assert e % (nw * CH) == 0
    ew_per = e // nw
    nch = ew_per // CH
    nvec = hd2 // L
    hv = 2
    nheads = nvec // hv
    wrows = 80
    assert n % wrows == 0
    nblk = n // wrows
    nblk_ceil = (nblk + NS - 1) // NS

    mesh = plsc.VectorSubcoreMesh(core_axis_name="c", subcore_axis_name="s")

    @functools.partial(
        pl.kernel,
        out_type=(jax.ShapeDtypeStruct((2 * n, hd2), jnp.float32),
                  jax.ShapeDtypeStruct((2 * n, L), jnp.float32)),
        mesh=mesh,
        compiler_params=pltpu.CompilerParams(use_tc_tiling_on_sc=False),
        scratch_types=dict(
            acc_sh=pltpu.VMEM_SHARED((n, hd2), jnp.float32),
            accd_sh=pltpu.VMEM_SHARED((n, L), jnp.float32),
            idxs=pltpu.VMEM((CH,), jnp.int32),
            idxd=pltpu.VMEM((CH,), jnp.int32),
            ewv=pltpu.VMEM((CH,), jnp.float32),
            xlr=pltpu.VMEM((CH, hd2), jnp.float32),
            xrr=pltpu.VMEM((CH, hd2), jnp.float32),
            stage=pltpu.VMEM((CH, hd2), jnp.float32),
            staged=pltpu.VMEM((CH, L), jnp.float32),
            wei=pltpu.VMEM((hd2,), jnp.float32),
            atti=pltpu.VMEM((hd2,), jnp.float32),
            sem1=pltpu.SemaphoreType.DMA,
            sem2=pltpu.SemaphoreType.DMA,
        ),
    )
    def edge_kernel(xl_hbm, xr_hbm, src_hbm, dst_hbm, ew_hbm, we_hbm,
                    att_hbm, out_hbm, outd_hbm, acc_sh, accd_sh, idxs, idxd,
                    ewv, xlr, xrr, stage, staged, wei, atti,
                    sem1, sem2):
        cid = lax.axis_index("c")
        sid = lax.axis_index("s")
        wid = sid * NC + cid

        pltpu.sync_copy(we_hbm, wei)
        pltpu.sync_copy(att_hbm, atti)

        zv = jnp.zeros((L,), jnp.float32)

        def zrow(r, _):
            for j in range(hd2 // L):
                stage[r, pl.ds(j * L, L)] = zv
            staged[r, pl.ds(0, L)] = zv
            return 0

        lax.fori_loop(0, CH, zrow, 0)

        def zblk(b, _):
            blk = b * NS + sid

            @pl.when(blk < nblk)
            def _():
                pltpu.sync_copy(stage, acc_sh.at[pl.ds(blk * wrows, wrows)])
                pltpu.sync_copy(staged, accd_sh.at[pl.ds(blk * wrows, wrows)])

            return 0

        lax.fori_loop(0, nblk_ceil, zblk, 0)
        plsc.subcore_barrier()

        lanes = lax.iota(jnp.int32, L)
        perms = [lanes ^ s for s in (8, 4, 2, 1)]
        base0 = wid * ew_per

        def chunk(c, _):
            base = base0 + c * CH
            pltpu.sync_copy(src_hbm.at[pl.ds(base, CH)], idxs)
            pltpu.sync_copy(dst_hbm.at[pl.ds(base, CH)], idxd)
            pltpu.sync_copy(ew_hbm.at[pl.ds(base, CH)], ewv)
            cp1 = pltpu.async_copy(xl_hbm.at[idxs], xlr, sem1)
            cp2 = pltpu.async_copy(xr_hbm.at[idxd], xrr, sem2)
            cp1.wait()
            cp2.wait()

            def egroup(g, _):
                ewg = ewv[pl.ds(g * L, L)]
                for u in range(L):
                    t = g * L + u
                    w = ewg[u]
                    den = jnp.zeros((L,), jnp.float32)
                    for h in range(nheads):
                        q = jnp.zeros((L,), jnp.float32)
                        xls = []
                        for k in range(hv):
                            i = h * hv + k
                            xv = xlr[t, pl.ds(i * L, L)]
                            z = (xv + xrr[t, pl.ds(i * L, L)]
                                 + w * wei[pl.ds(i * L, L)])
                            m = jnp.where(z >= 0.0, z, 0.2 * z)
                            q = q + m * atti[pl.ds(i * L, L)]
                            xls.append(xv)
                        for p in perms:
                            q = q + _vgather(q, p)
                        ex = jnp.exp(q)
                        for k in range(hv):
                            i = h * hv + k
                            stage[t, pl.ds(i * L, L)] = xls[k] * ex
                        den = jnp.where(lanes == h, ex, den)
                    staged[t, pl.ds(0, L)] = den
                return 0

            lax.fori_loop(0, CH // L, egroup, 0)
            pltpu.sync_copy(stage, acc_sh.at[idxd], add=True)
            pltpu.sync_copy(staged, accd_sh.at[idxd], add=True)
            return 0

        lax.fori_loop(0, nch, chunk, 0)
        plsc.subcore_barrier()

        def wblk(b, _):
            blk = b * NS + sid

            @pl.when(blk < nblk)
            def _():
                r = blk * wrows
                pltpu.sync_copy(acc_sh.at[pl.ds(r, wrows)],
                                out_hbm.at[pl.ds(cid * n + r, wrows)])
                pltpu.sync_copy(accd_sh.at[pl.ds(r, wrows)],
                                outd_hbm.at[pl.ds(cid * n + r, wrows)])

            return 0

        lax.fori_loop(0, nblk_ceil, wblk, 0)

    return edge_kernel


def _gat_layer(xp, src, dst, ew, Wl, bl, Wr, br, We, att, b):
    n = xp.shape[0]
    e = src.shape[0]
    heads, c = att.shape
    hd2 = heads * c
    xl = xp @ Wl + bl
    xr = xp @ Wr + br
    wev = We.reshape(hd2)
    attv = att.reshape(hd2)
    ek = _make_edge_kernel(n, e, hd2)
    acc, accd = ek(xl, xr, src, dst, ew, wev, attv)
    den = (accd[:n] + accd[n:])[:, :heads]
    num = (acc[:n] + acc[n:]).reshape(n, heads, c)
    out = (num / (den[:, :, None] + 1e-16)).mean(axis=1) + b
    return jax.nn.relu(out)


def _make_temporal_kernel(npad, t_len, hid, dk, heads, blk):
    nblk = npad // blk
    hd = dk // heads
    scale = hd ** -0.5

    def body(xsT, xpT, wqT, bq, wkT, bk, wvT, bv, woT, bo,
             fw1T, fb1, fw2T, fb2, rw1T, rb1, rw2T, rb2,
             toutT, fout, rout, q_s, k_s, v_s, srow_s):
        xp = xpT[...]

        def qkv(t, _):
            z = xsT[t] + xp
            q_s[t] = jnp.dot(wqT[...], z,
                             preferred_element_type=jnp.float32) + bq[...]
            k_s[t] = jnp.dot(wkT[...], z,
                             preferred_element_type=jnp.float32) + bk[...]
            v_s[t] = jnp.dot(wvT[...], z,
                             preferred_element_type=jnp.float32) + bv[...]
            return 0

        lax.fori_loop(0, t_len, qkv, 0)

        def attend(t, _):
            qt = q_s[t] * scale

            def score(u, _):
                prod = (qt * k_s[u]).reshape(heads, hd, blk)
                srow_s[u] = jnp.sum(prod, axis=1)
                return 0

            lax.fori_loop(0, t_len, score, 0)
            s = srow_s[...]
            m = jnp.max(s, axis=0, keepdims=True)
            e = jnp.exp(s - m)
            srow_s[...] = e / jnp.sum(e, axis=0, keepdims=True)

            def accum(u, c):
                return c + srow_s[u][:, None, :] * v_s[u].reshape(
                    heads, hd, blk)

            ctx = lax.fori_loop(
                0, t_len, accum, jnp.zeros((heads, hd, blk), jnp.float32))
            toutT[t] = (jnp.dot(woT[...], ctx.reshape(dk, blk),
                                preferred_element_type=jnp.float32) + bo[...])
            return 0

        lax.fori_loop(0, t_len, attend, 0)

        last = toutT[t_len - 1]
        h1 = jax.nn.relu(jnp.dot(fw1T[...], last,
                                 preferred_element_type=jnp.float32)
                         + fb1[...])
        fout[...] = jax.nn.relu(jnp.dot(fw2T[...], h1,
                                        preferred_element_type=jnp.float32)
                                + fb2[...])
        h2 = jax.nn.relu(jnp.dot(rw1T[...], last,
                                 preferred_element_type=jnp.float32)
                         + rb1[...])
        rout[...] = jax.nn.sigmoid(jnp.dot(rw2T[...], h2,
                                           preferred_element_type=jnp.float32)
                                   + rb2[...])

    full = lambda shape: pl.BlockSpec(shape, lambda i: (0,) * len(shape))
    return pl.pallas_call(
        body,
        grid=(nblk,),
        in_specs=[
            pl.BlockSpec((t_len, hid, blk), lambda i: (0, 0, i)),
            pl.BlockSpec((hid, blk), lambda i: (0, i)),
            full((dk, hid)), full((dk, 1)),
            full((dk, hid)), full((dk, 1)),
            full((dk, hid)), full((dk, 1)),
            full((hid, dk)), full((hid, 1)),
            full((hid, hid)), full((hid, 1)),
            full((1, hid)), full((1, 1)),
            full((hid, hid)), full((hid, 1)),
            full((1, hid)), full((1, 1)),
        ],
        out_specs=[
            pl.BlockSpec((t_len, hid, blk), lambda i: (0, 0, i)),
            pl.BlockSpec((1, blk), lambda i: (0, i)),
            pl.BlockSpec((1, blk), lambda i: (0, i)),
        ],
        out_shape=[
            jax.ShapeDtypeStruct((t_len, hid, npad), jnp.float32),
            jax.ShapeDtypeStruct((1, npad), jnp.float32),
            jax.ShapeDtypeStruct((1, npad), jnp.float32),
        ],
        scratch_shapes=[
            pltpu.VMEM((t_len, dk, blk), jnp.float32),
            pltpu.VMEM((t_len, dk, blk), jnp.float32),
            pltpu.VMEM((t_len, dk, blk), jnp.float32),
            pltpu.VMEM((t_len, heads, blk), jnp.float32),
        ],
        compiler_params=pltpu.CompilerParams(
            dimension_semantics=("arbitrary",)),
    )


def kernel(x, edge_index, edge_weight, Wp, bp, g0_Wl, g0_bl, g0_Wr, g0_br,
           g0_We, g0_att, g0_b, g1_Wl, g1_bl, g1_Wr, g1_br, g1_We, g1_att,
           g1_b, Wq, bq, Wk, bk, Wv, bv, Wo, bo, f_W1, f_b1, f_W2, f_b2,
           r_W1, r_b1, r_W2, r_b2):
    heads = g0_att.shape[0]
    src = edge_index[0]
    dst = edge_index[1]
    x_seq = x @ Wp + bp
    x_pool = x_seq.mean(axis=1)
    x_pool = _gat_layer(x_pool, src, dst, edge_weight,
                        g0_Wl, g0_bl, g0_Wr, g0_br, g0_We, g0_att, g0_b)
    x_pool = _gat_layer(x_pool, src, dst, edge_weight,
                        g1_Wl, g1_bl, g1_Wr, g1_br, g1_We, g1_att, g1_b)
    n, t_len, _ = x.shape
    hid = Wp.shape[1]
    dk = Wq.shape[1]
    blk = 512
    npad = -(-n // blk) * blk
    xsT = jnp.transpose(x_seq, (1, 2, 0))
    xsT = jnp.pad(xsT, ((0, 0), (0, 0), (0, npad - n)))
    xpT = jnp.pad(x_pool.T, ((0, 0), (0, npad - n)))
    tk = _make_temporal_kernel(npad, t_len, hid, dk, heads, blk)
    toutT, fout, rout = tk(
        xsT, xpT, Wq.T, bq[:, None], Wk.T, bk[:, None], Wv.T, bv[:, None],
        Wo.T, bo[:, None], f_W1.T, f_b1[:, None], f_W2.T, f_b2[:, None],
        r_W1.T, r_b1[:, None], r_W2.T, r_b2[:, None])
    temporal_out = jnp.transpose(toutT[:, :, :n], (0, 2, 1))
    forecast = fout[0, :n, None]
    risk = rout[0, :n, None]
    return (forecast, risk, temporal_out)


# speedup vs baseline: 66.7189x; 3.6958x over previous
import functools
import jax
import jax.numpy as jnp
from jax import lax
from jax.experimental import pallas as pl
from jax.experimental.pallas import tpu as pltpu
from jax.experimental.pallas import tpu_sc as plsc

NC = 2
NS = 16
L = 16
CH = 40


def _vgather(x, idx):
    dn = lax.GatherDimensionNumbers(
        offset_dims=(), collapsed_slice_dims=(0,), start_index_map=(0,))
    return lax.gather(x, idx[:, None], dn, (1,),
                      mode=lax.GatherScatterMode.PROMISE_IN_BOUNDS)


def _make_edge_kernel(n, e, hd2):
    nw = NC * NS
    assert e % (nw * CH) == 0
    ew_per = e // nw
    nch = ew_per // CH
    npairs = nch // 2
    assert nch % 2 == 0
    nvec = hd2 // L
    hv = 2
    nheads = nvec // hv
    wrows = CH
    assert n % wrows == 0
    nblk = n // wrows
    nblk_ceil = (nblk + NS - 1) // NS

    mesh = plsc.VectorSubcoreMesh(core_axis_name="c", subcore_axis_name="s")

    @functools.partial(
        pl.kernel,
        out_type=(jax.ShapeDtypeStruct((2 * n, hd2), jnp.float32),
                  jax.ShapeDtypeStruct((2 * n, L), jnp.float32)),
        mesh=mesh,
        compiler_params=pltpu.CompilerParams(use_tc_tiling_on_sc=False,
                                             needs_layout_passes=False),
        scratch_types=dict(
            acc_sh=pltpu.VMEM_SHARED((n, hd2), jnp.float32),
            accd_sh=pltpu.VMEM_SHARED((n, L), jnp.float32),
            ebuf0=pltpu.VMEM((4, CH), jnp.int32),
            ebuf1=pltpu.VMEM((4, CH), jnp.int32),
            sdst0=pltpu.VMEM((CH,), jnp.int32),
            sdst1=pltpu.VMEM((CH,), jnp.int32),
            xlr0=pltpu.VMEM((CH, hd2), jnp.float32),
            xlr1=pltpu.VMEM((CH, hd2), jnp.float32),
            xrr0=pltpu.VMEM((CH, hd2), jnp.float32),
            xrr1=pltpu.VMEM((CH, hd2), jnp.float32),
            stage0=pltpu.VMEM((CH, hd2), jnp.float32),
            stage1=pltpu.VMEM((CH, hd2), jnp.float32),
            staged0=pltpu.VMEM((CH, L), jnp.float32),
            staged1=pltpu.VMEM((CH, L), jnp.float32),
            wei=pltpu.VMEM((hd2,), jnp.float32),
            atti=pltpu.VMEM((hd2,), jnp.float32),
            gxl0=pltpu.SemaphoreType.DMA, gxl1=pltpu.SemaphoreType.DMA,
            gxr0=pltpu.SemaphoreType.DMA, gxr1=pltpu.SemaphoreType.DMA,
            isem0=pltpu.SemaphoreType.DMA, isem1=pltpu.SemaphoreType.DMA,
            ssem0=pltpu.SemaphoreType.DMA, ssem1=pltpu.SemaphoreType.DMA,
            ssn0=pltpu.SemaphoreType.DMA, ssn1=pltpu.SemaphoreType.DMA,
        ),
    )
    def edge_kernel(xl_hbm, xr_hbm, ep_hbm, we_hbm, att_hbm,
                    out_hbm, outd_hbm, acc_sh, accd_sh,
                    ebuf0, ebuf1, sdst0, sdst1, xlr0, xlr1, xrr0, xrr1,
                    stage0, stage1, staged0, staged1, wei, atti,
                    gxl0, gxl1, gxr0, gxr1, isem0, isem1,
                    ssem0, ssem1, ssn0, ssn1):
        cid = lax.axis_index("c")
        sid = lax.axis_index("s")
        wid = sid * NC + cid

        pltpu.sync_copy(we_hbm, wei)
        pltpu.sync_copy(att_hbm, atti)
        wv = [wei[pl.ds(i * L, L)] for i in range(nvec)]
        av = [atti[pl.ds(i * L, L)] for i in range(nvec)]
        lanes = lax.iota(jnp.int32, L)
        perms = [lanes ^ s for s in (8, 4, 2, 1)]
        hmask = [lanes == h for h in range(nheads)]

        zv = jnp.zeros((L,), jnp.float32)

        def zrow(r, _):
            for j in range(hd2 // L):
                stage0[r, pl.ds(j * L, L)] = zv
            staged0[r, pl.ds(0, L)] = zv
            return 0

        lax.fori_loop(0, CH, zrow, 0)

        def zblk(b, _):
            blk = b * NS + sid

            @pl.when(blk < nblk)
            def _():
                pltpu.sync_copy(stage0, acc_sh.at[pl.ds(blk * wrows, wrows)])
                pltpu.sync_copy(staged0,
                                accd_sh.at[pl.ds(blk * wrows, wrows)])

            return 0

        lax.fori_loop(0, nblk_ceil, zblk, 0)
        plsc.subcore_barrier()

        base0 = wid * ew_per
        par_refs = [
            (ebuf0, sdst0, xlr0, xrr0, stage0, staged0,
             gxl0, gxr0, isem0, ssem0, ssn0),
            (ebuf1, sdst1, xlr1, xrr1, stage1, staged1,
             gxl1, gxr1, isem1, ssem1, ssn1),
        ]

        for par in range(2):
            ebuf, _, xlr, xrr = par_refs[par][:4]
            gxl, gxr = par_refs[par][6:8]
            pltpu.sync_copy(ep_hbm.at[:, pl.ds(base0 + par * CH, CH)], ebuf)
            pltpu.async_copy(xl_hbm.at[ebuf.at[0]], xlr, gxl)
            pltpu.async_copy(xr_hbm.at[ebuf.at[1]], xrr, gxr)

        def compute_chunk(ebuf, xlr, xrr, stage, staged):
            def egroup(g, _):
                ewg = plsc.bitcast(ebuf[2, pl.ds(g * 8, L)], jnp.float32)
                for u in range(8):
                    t = g * 8 + u
                    w = ewg[u]
                    den = jnp.zeros((L,), jnp.float32)
                    for h in range(nheads):
                        q = jnp.zeros((L,), jnp.float32)
                        xls = []
                        for k in range(hv):
                            i = h * hv + k
                            xv = xlr[t, pl.ds(i * L, L)]
                            z = xv + xrr[t, pl.ds(i * L, L)] + w * wv[i]
                            m = jnp.where(z >= 0.0, z, 0.2 * z)
                            q = q + m * av[i]
                            xls.append(xv)
                        for p in perms:
                            q = q + _vgather(q, p)
                        ex = jnp.exp(q)
                        for k in range(hv):
                            i = h * hv + k
                            stage[t, pl.ds(i * L, L)] = xls[k] * ex
                        den = jnp.where(hmask[h], ex, den)
                    staged[t, pl.ds(0, L)] = den
                return 0

            lax.fori_loop(0, CH // 8, egroup, 0)

        def pair(j, _):
            for par in range(2):
                (ebuf, sdst, xlr, xrr, stage, staged,
                 gxl, gxr, isem, ssem, ssn) = par_refs[par]
                c = 2 * j + par
                base = base0 + c * CH
                pltpu.make_async_copy(xl_hbm.at[ebuf.at[0]], xlr, gxl).wait()
                pltpu.make_async_copy(xr_hbm.at[ebuf.at[1]], xrr, gxr).wait()

                @pl.when(j >= 1)
                def _():
                    pltpu.make_async_copy(stage, acc_sh.at[sdst], ssem).wait()
                    pltpu.make_async_copy(staged, accd_sh.at[sdst],
                                          ssn).wait()

                pltpu.async_copy(ep_hbm.at[1, pl.ds(base, CH)], sdst, isem)

                compute_chunk(ebuf, xlr, xrr, stage, staged)

                @pl.when(j < npairs - 1)
                def _():
                    pltpu.sync_copy(
                        ep_hbm.at[:, pl.ds(base + 2 * CH, CH)], ebuf)
                    pltpu.async_copy(xl_hbm.at[ebuf.at[0]], xlr, gxl)
                    pltpu.async_copy(xr_hbm.at[ebuf.at[1]], xrr, gxr)

                pltpu.make_async_copy(ep_hbm.at[1, pl.ds(base, CH)], sdst,
                                      isem).wait()
                pltpu.async_copy(stage, acc_sh.at[sdst], ssem, add=True)
                pltpu.async_copy(staged, accd_sh.at[sdst], ssn, add=True)
            return 0

        lax.fori_loop(0, npairs, pair, 0)
        for par in range(2):
            (_, sdst, _, _, stage, staged,
             _, _, _, ssem, ssn) = par_refs[par]
            pltpu.make_async_copy(stage, acc_sh.at[sdst], ssem).wait()
            pltpu.make_async_copy(staged, accd_sh.at[sdst], ssn).wait()
        plsc.subcore_barrier()

        def wblk(b, _):
            blk = b * NS + sid

            @pl.when(blk < nblk)
            def _():
                r = blk * wrows
                pltpu.sync_copy(acc_sh.at[pl.ds(r, wrows)],
                                out_hbm.at[pl.ds(cid * n + r, wrows)])
                pltpu.sync_copy(accd_sh.at[pl.ds(r, wrows)],
                                outd_hbm.at[pl.ds(cid * n + r, wrows)])

            return 0

        lax.fori_loop(0, nblk_ceil, wblk, 0)

    return edge_kernel


def _gat_layer(xp, src, dst, ew, Wl, bl, Wr, br, We, att, b):
    n = xp.shape[0]
    e = src.shape[0]
    heads, c = att.shape
    hd2 = heads * c
    xl = xp @ Wl + bl
    xr = xp @ Wr + br
    epack = jnp.concatenate(
        [src[None], dst[None],
         jax.lax.bitcast_convert_type(ew, jnp.int32)[None],
         jnp.zeros((1, e), jnp.int32)], axis=0)
    wev = We.reshape(hd2)
    attv = att.reshape(hd2)
    ek = _make_edge_kernel(n, e, hd2)
    acc, accd = ek(xl, xr, epack, wev, attv)
    den = (accd[:n] + accd[n:])[:, :heads]
    num = (acc[:n] + acc[n:]).reshape(n, heads, c)
    out = (num / (den[:, :, None] + 1e-16)).mean(axis=1) + b
    return jax.nn.relu(out)


def _make_temporal_kernel(npad, t_len, hid, dk, heads, blk):
    nblk = npad // blk
    hd = dk // heads
    scale = hd ** -0.5

    def body(xsT, xpT, wqT, bq, wkT, bk, wvT, bv, woT, bo,
             fw1T, fb1, fw2T, fb2, rw1T, rb1, rw2T, rb2,
             toutT, fout, rout, q_s, k_s, v_s, srow_s):
        xp = xpT[...]

        def qkv(t, _):
            z = xsT[t] + xp
            q_s[t] = jnp.dot(wqT[...], z,
                             preferred_element_type=jnp.float32) + bq[...]
            k_s[t] = jnp.dot(wkT[...], z,
                             preferred_element_type=jnp.float32) + bk[...]
            v_s[t] = jnp.dot(wvT[...], z,
                             preferred_element_type=jnp.float32) + bv[...]
            return 0

        lax.fori_loop(0, t_len, qkv, 0)

        def attend(t, _):
            qt = q_s[t] * scale

            def score(u, _):
                prod = (qt * k_s[u]).reshape(heads, hd, blk)
                srow_s[u] = jnp.sum(prod, axis=1)
                return 0

            lax.fori_loop(0, t_len, score, 0)
            s = srow_s[...]
            m = jnp.max(s, axis=0, keepdims=True)
            e = jnp.exp(s - m)
            srow_s[...] = e / jnp.sum(e, axis=0, keepdims=True)

            def accum(u, c):
                return c + srow_s[u][:, None, :] * v_s[u].reshape(
                    heads, hd, blk)

            ctx = lax.fori_loop(
                0, t_len, accum, jnp.zeros((heads, hd, blk), jnp.float32))
            toutT[t] = (jnp.dot(woT[...], ctx.reshape(dk, blk),
                                preferred_element_type=jnp.float32) + bo[...])
            return 0

        lax.fori_loop(0, t_len, attend, 0)

        last = toutT[t_len - 1]
        h1 = jax.nn.relu(jnp.dot(fw1T[...], last,
                                 preferred_element_type=jnp.float32)
                         + fb1[...])
        fout[...] = jax.nn.relu(jnp.dot(fw2T[...], h1,
                                        preferred_element_type=jnp.float32)
                                + fb2[...])
        h2 = jax.nn.relu(jnp.dot(rw1T[...], last,
                                 preferred_element_type=jnp.float32)
                         + rb1[...])
        rout[...] = jax.nn.sigmoid(jnp.dot(rw2T[...], h2,
                                           preferred_element_type=jnp.float32)
                                   + rb2[...])

    full = lambda shape: pl.BlockSpec(shape, lambda i: (0,) * len(shape))
    return pl.pallas_call(
        body,
        grid=(nblk,),
        in_specs=[
            pl.BlockSpec((t_len, hid, blk), lambda i: (0, 0, i)),
            pl.BlockSpec((hid, blk), lambda i: (0, i)),
            full((dk, hid)), full((dk, 1)),
            full((dk, hid)), full((dk, 1)),
            full((dk, hid)), full((dk, 1)),
            full((hid, dk)), full((hid, 1)),
            full((hid, hid)), full((hid, 1)),
            full((1, hid)), full((1, 1)),
            full((hid, hid)), full((hid, 1)),
            full((1, hid)), full((1, 1)),
        ],
        out_specs=[
            pl.BlockSpec((t_len, hid, blk), lambda i: (0, 0, i)),
            pl.BlockSpec((1, blk), lambda i: (0, i)),
            pl.BlockSpec((1, blk), lambda i: (0, i)),
        ],
        out_shape=[
            jax.ShapeDtypeStruct((t_len, hid, npad), jnp.float32),
            jax.ShapeDtypeStruct((1, npad), jnp.float32),
            jax.ShapeDtypeStruct((1, npad), jnp.float32),
        ],
        scratch_shapes=[
            pltpu.VMEM((t_len, dk, blk), jnp.float32),
            pltpu.VMEM((t_len, dk, blk), jnp.float32),
            pltpu.VMEM((t_len, dk, blk), jnp.float32),
            pltpu.VMEM((t_len, heads, blk), jnp.float32),
        ],
        compiler_params=pltpu.CompilerParams(
            dimension_semantics=("arbitrary",)),
    )


def kernel(x, edge_index, edge_weight, Wp, bp, g0_Wl, g0_bl, g0_Wr, g0_br,
           g0_We, g0_att, g0_b, g1_Wl, g1_bl, g1_Wr, g1_br, g1_We, g1_att,
           g1_b, Wq, bq, Wk, bk, Wv, bv, Wo, bo, f_W1, f_b1, f_W2, f_b2,
           r_W1, r_b1, r_W2, r_b2):
    heads = g0_att.shape[0]
    src = edge_index[0]
    dst = edge_index[1]
    x_seq = x @ Wp + bp
    x_pool = x_seq.mean(axis=1)
    x_pool = _gat_layer(x_pool, src, dst, edge_weight,
                        g0_Wl, g0_bl, g0_Wr, g0_br, g0_We, g0_att, g0_b)
    x_pool = _gat_layer(x_pool, src, dst, edge_weight,
                        g1_Wl, g1_bl, g1_Wr, g1_br, g1_We, g1_att, g1_b)
    n, t_len, _ = x.shape
    hid = Wp.shape[1]
    dk = Wq.shape[1]
    blk = 512
    npad = -(-n // blk) * blk
    xsT = jnp.transpose(x_seq, (1, 2, 0))
    xsT = jnp.pad(xsT, ((0, 0), (0, 0), (0, npad - n)))
    xpT = jnp.pad(x_pool.T, ((0, 0), (0, npad - n)))
    tk = _make_temporal_kernel(npad, t_len, hid, dk, heads, blk)
    toutT, fout, rout = tk(
        xsT, xpT, Wq.T, bq[:, None], Wk.T, bk[:, None], Wv.T, bv[:, None],
        Wo.T, bo[:, None], f_W1.T, f_b1[:, None], f_W2.T, f_b2[:, None],
        r_W1.T, r_b1[:, None], r_W2.T, r_b2[:, None])
    temporal_out = jnp.transpose(toutT[:, :, :n], (0, 2, 1))
    forecast = fout[0, :n, None]
    risk = rout[0, :n, None]
    return (forecast, risk, temporal_out)


# speedup vs baseline: 84.3224x; 1.2638x over previous
import functools
import jax
import jax.numpy as jnp
from jax import lax
from jax.experimental import pallas as pl
from jax.experimental.pallas import tpu as pltpu
from jax.experimental.pallas import tpu_sc as plsc

NC = 2
NS = 16
L = 16
CH = 40


def _vgather(x, idx):
    dn = lax.GatherDimensionNumbers(
        offset_dims=(), collapsed_slice_dims=(0,), start_index_map=(0,))
    return lax.gather(x, idx[:, None], dn, (1,),
                      mode=lax.GatherScatterMode.PROMISE_IN_BOUNDS)


def _make_edge_kernel(n, e, hd2):
    nw = NC * NS
    assert e % (nw * CH) == 0
    ew_per = e // nw
    nch = ew_per // CH
    npairs = nch // 2
    assert nch % 2 == 0
    nvec = hd2 // L
    hv = 2
    nheads = nvec // hv
    wrows = CH
    assert n % wrows == 0
    nblk = n // wrows
    nblk_ceil = (nblk + NS - 1) // NS

    mesh = plsc.VectorSubcoreMesh(core_axis_name="c", subcore_axis_name="s")

    @functools.partial(
        pl.kernel,
        out_type=(jax.ShapeDtypeStruct((2 * n, hd2), jnp.float32),
                  jax.ShapeDtypeStruct((2 * n, L), jnp.float32)),
        mesh=mesh,
        compiler_params=pltpu.CompilerParams(use_tc_tiling_on_sc=False,
                                             needs_layout_passes=False),
        scratch_types=dict(
            acc_sh=pltpu.VMEM_SHARED((n, hd2), jnp.float32),
            accd_sh=pltpu.VMEM_SHARED((n, L), jnp.float32),
            ebuf0=pltpu.VMEM((4, CH), jnp.int32),
            ebuf1=pltpu.VMEM((4, CH), jnp.int32),
            sdst0=pltpu.VMEM((CH,), jnp.int32),
            sdst1=pltpu.VMEM((CH,), jnp.int32),
            xlr0=pltpu.VMEM((CH, hd2), jnp.float32),
            xlr1=pltpu.VMEM((CH, hd2), jnp.float32),
            xrr0=pltpu.VMEM((CH, hd2), jnp.float32),
            xrr1=pltpu.VMEM((CH, hd2), jnp.float32),
            stage0=pltpu.VMEM((CH, hd2), jnp.float32),
            stage1=pltpu.VMEM((CH, hd2), jnp.float32),
            staged0=pltpu.VMEM((CH, L), jnp.float32),
            staged1=pltpu.VMEM((CH, L), jnp.float32),
            wei=pltpu.VMEM((hd2,), jnp.float32),
            atti=pltpu.VMEM((hd2,), jnp.float32),
            gxl0=pltpu.SemaphoreType.DMA, gxl1=pltpu.SemaphoreType.DMA,
            gxr0=pltpu.SemaphoreType.DMA, gxr1=pltpu.SemaphoreType.DMA,
            isem0=pltpu.SemaphoreType.DMA, isem1=pltpu.SemaphoreType.DMA,
            ssem0=pltpu.SemaphoreType.DMA, ssem1=pltpu.SemaphoreType.DMA,
            ssn0=pltpu.SemaphoreType.DMA, ssn1=pltpu.SemaphoreType.DMA,
        ),
    )
    def edge_kernel(xl_hbm, xr_hbm, ep_hbm, we_hbm, att_hbm,
                    out_hbm, outd_hbm, acc_sh, accd_sh,
                    ebuf0, ebuf1, sdst0, sdst1, xlr0, xlr1, xrr0, xrr1,
                    stage0, stage1, staged0, staged1, wei, atti,
                    gxl0, gxl1, gxr0, gxr1, isem0, isem1,
                    ssem0, ssem1, ssn0, ssn1):
        cid = lax.axis_index("c")
        sid = lax.axis_index("s")
        wid = sid * NC + cid

        pltpu.sync_copy(we_hbm, wei)
        pltpu.sync_copy(att_hbm, atti)
        wv = [wei[pl.ds(i * L, L)] for i in range(nvec)]
        av = [atti[pl.ds(i * L, L)] for i in range(nvec)]
        lanes = lax.iota(jnp.int32, L)
        perms = [lanes ^ s for s in (8, 4, 2, 1)]
        hmask = [lanes == h for h in range(nheads)]

        zv = jnp.zeros((L,), jnp.float32)

        def zrow(r, _):
            for j in range(hd2 // L):
                stage0[r, pl.ds(j * L, L)] = zv
            staged0[r, pl.ds(0, L)] = zv
            return 0

        lax.fori_loop(0, CH, zrow, 0)

        def zblk(b, _):
            blk = b * NS + sid

            @pl.when(blk < nblk)
            def _():
                pltpu.sync_copy(stage0, acc_sh.at[pl.ds(blk * wrows, wrows)])
                pltpu.sync_copy(staged0,
                                accd_sh.at[pl.ds(blk * wrows, wrows)])

            return 0

        lax.fori_loop(0, nblk_ceil, zblk, 0)
        plsc.subcore_barrier()

        base0 = wid * ew_per
        par_refs = [
            (ebuf0, sdst0, xlr0, xrr0, stage0, staged0,
             gxl0, gxr0, isem0, ssem0, ssn0),
            (ebuf1, sdst1, xlr1, xrr1, stage1, staged1,
             gxl1, gxr1, isem1, ssem1, ssn1),
        ]

        for par in range(2):
            ebuf, _, xlr, xrr = par_refs[par][:4]
            gxl, gxr = par_refs[par][6:8]
            pltpu.sync_copy(ep_hbm.at[:, pl.ds(base0 + par * CH, CH)], ebuf)
            pltpu.async_copy(xl_hbm.at[ebuf.at[0]], xlr, gxl)
            pltpu.async_copy(xr_hbm.at[ebuf.at[1]], xrr, gxr)

        def compute_chunk(ebuf, xlr, xrr, stage, staged):
            def egroup(g, _):
                ewg = plsc.bitcast(ebuf[2, pl.ds(g * 8, L)], jnp.float32)
                for u0 in range(0, 8, 4):
                    ts = [g * 8 + u0 + i for i in range(4)]
                    qs = []
                    for i, t in enumerate(ts):
                        w = ewg[u0 + i]
                        qh = []
                        for h in range(nheads):
                            q = jnp.zeros((L,), jnp.float32)
                            for k in range(hv):
                                idx = h * hv + k
                                z = (xlr[t, pl.ds(idx * L, L)]
                                     + xrr[t, pl.ds(idx * L, L)]
                                     + w * wv[idx])
                                m = jnp.where(z >= 0.0, z, 0.2 * z)
                                q = q + m * av[idx]
                            qh.append(q)
                        qs.append(qh)
                    for p in perms:
                        qs = [[q + _vgather(q, p) for q in qh] for qh in qs]
                    for i, t in enumerate(ts):
                        den = jnp.zeros((L,), jnp.float32)
                        for h in range(nheads):
                            ex = jnp.exp(qs[i][h])
                            for k in range(hv):
                                idx = h * hv + k
                                stage[t, pl.ds(idx * L, L)] = (
                                    xlr[t, pl.ds(idx * L, L)] * ex)
                            den = jnp.where(hmask[h], ex, den)
                        staged[t, pl.ds(0, L)] = den
                return 0

            lax.fori_loop(0, CH // 8, egroup, 0)

        def pair(j, _):
            for par in range(2):
                (ebuf, sdst, xlr, xrr, stage, staged,
                 gxl, gxr, isem, ssem, ssn) = par_refs[par]
                c = 2 * j + par
                base = base0 + c * CH
                pltpu.make_async_copy(xl_hbm.at[ebuf.at[0]], xlr, gxl).wait()
                pltpu.make_async_copy(xr_hbm.at[ebuf.at[1]], xrr, gxr).wait()

                @pl.when(j >= 1)
                def _():
                    pltpu.make_async_copy(stage, acc_sh.at[sdst], ssem).wait()
                    pltpu.make_async_copy(staged, accd_sh.at[sdst],
                                          ssn).wait()

                pltpu.async_copy(ep_hbm.at[1, pl.ds(base, CH)], sdst, isem)

                compute_chunk(ebuf, xlr, xrr, stage, staged)

                @pl.when(j < npairs - 1)
                def _():
                    pltpu.sync_copy(
                        ep_hbm.at[:, pl.ds(base + 2 * CH, CH)], ebuf)
                    pltpu.async_copy(xl_hbm.at[ebuf.at[0]], xlr, gxl)
                    pltpu.async_copy(xr_hbm.at[ebuf.at[1]], xrr, gxr)

                pltpu.make_async_copy(ep_hbm.at[1, pl.ds(base, CH)], sdst,
                                      isem).wait()
                pltpu.async_copy(stage, acc_sh.at[sdst], ssem, add=True)
                pltpu.async_copy(staged, accd_sh.at[sdst], ssn, add=True)
            return 0

        lax.fori_loop(0, npairs, pair, 0)
        for par in range(2):
            (_, sdst, _, _, stage, staged,
             _, _, _, ssem, ssn) = par_refs[par]
            pltpu.make_async_copy(stage, acc_sh.at[sdst], ssem).wait()
            pltpu.make_async_copy(staged, accd_sh.at[sdst], ssn).wait()
        plsc.subcore_barrier()

        def wblk(b, _):
            blk = b * NS + sid

            @pl.when(blk < nblk)
            def _():
                r = blk * wrows
                pltpu.sync_copy(acc_sh.at[pl.ds(r, wrows)],
                                out_hbm.at[pl.ds(cid * n + r, wrows)])
                pltpu.sync_copy(accd_sh.at[pl.ds(r, wrows)],
                                outd_hbm.at[pl.ds(cid * n + r, wrows)])

            return 0

        lax.fori_loop(0, nblk_ceil, wblk, 0)

    return edge_kernel


def _gat_layer(xp, src, dst, ew, Wl, bl, Wr, br, We, att, b):
    n = xp.shape[0]
    e = src.shape[0]
    heads, c = att.shape
    hd2 = heads * c
    xl = xp @ Wl + bl
    xr = xp @ Wr + br
    epack = jnp.concatenate(
        [src[None], dst[None],
         jax.lax.bitcast_convert_type(ew, jnp.int32)[None],
         jnp.zeros((1, e), jnp.int32)], axis=0)
    wev = We.reshape(hd2)
    attv = att.reshape(hd2)
    ek = _make_edge_kernel(n, e, hd2)
    acc, accd = ek(xl, xr, epack, wev, attv)
    den = (accd[:n] + accd[n:])[:, :heads]
    num = (acc[:n] + acc[n:]).reshape(n, heads, c)
    out = (num / (den[:, :, None] + 1e-16)).mean(axis=1) + b
    return jax.nn.relu(out)


def _make_temporal_kernel(npad, t_len, hid, dk, heads, blk):
    nblk = npad // blk
    hd = dk // heads
    scale = hd ** -0.5

    def body(xsT, xpT, wqT, bq, wkT, bk, wvT, bv, woT, bo,
             fw1T, fb1, fw2T, fb2, rw1T, rb1, rw2T, rb2,
             toutT, fout, rout, q_s, k_s, v_s, srow_s):
        xp = xpT[...]

        def qkv(t, _):
            z = xsT[t] + xp
            q_s[t] = jnp.dot(wqT[...], z,
                             preferred_element_type=jnp.float32) + bq[...]
            k_s[t] = jnp.dot(wkT[...], z,
                             preferred_element_type=jnp.float32) + bk[...]
            v_s[t] = jnp.dot(wvT[...], z,
                             preferred_element_type=jnp.float32) + bv[...]
            return 0

        lax.fori_loop(0, t_len, qkv, 0)

        def attend(t, _):
            qt = q_s[t] * scale

            def score(u, _):
                prod = (qt * k_s[u]).reshape(heads, hd, blk)
                srow_s[u] = jnp.sum(prod, axis=1)
                return 0

            lax.fori_loop(0, t_len, score, 0)
            s = srow_s[...]
            m = jnp.max(s, axis=0, keepdims=True)
            e = jnp.exp(s - m)
            srow_s[...] = e / jnp.sum(e, axis=0, keepdims=True)

            def accum(u, c):
                return c + srow_s[u][:, None, :] * v_s[u].reshape(
                    heads, hd, blk)

            ctx = lax.fori_loop(
                0, t_len, accum, jnp.zeros((heads, hd, blk), jnp.float32))
            toutT[t] = (jnp.dot(woT[...], ctx.reshape(dk, blk),
                                preferred_element_type=jnp.float32) + bo[...])
            return 0

        lax.fori_loop(0, t_len, attend, 0)

        last = toutT[t_len - 1]
        h1 = jax.nn.relu(jnp.dot(fw1T[...], last,
                                 preferred_element_type=jnp.float32)
                         + fb1[...])
        fout[...] = jax.nn.relu(jnp.dot(fw2T[...], h1,
                                        preferred_element_type=jnp.float32)
                                + fb2[...])
        h2 = jax.nn.relu(jnp.dot(rw1T[...], last,
                                 preferred_element_type=jnp.float32)
                         + rb1[...])
        rout[...] = jax.nn.sigmoid(jnp.dot(rw2T[...], h2,
                                           preferred_element_type=jnp.float32)
                                   + rb2[...])

    full = lambda shape: pl.BlockSpec(shape, lambda i: (0,) * len(shape))
    return pl.pallas_call(
        body,
        grid=(nblk,),
        in_specs=[
            pl.BlockSpec((t_len, hid, blk), lambda i: (0, 0, i)),
            pl.BlockSpec((hid, blk), lambda i: (0, i)),
            full((dk, hid)), full((dk, 1)),
            full((dk, hid)), full((dk, 1)),
            full((dk, hid)), full((dk, 1)),
            full((hid, dk)), full((hid, 1)),
            full((hid, hid)), full((hid, 1)),
            full((1, hid)), full((1, 1)),
            full((hid, hid)), full((hid, 1)),
            full((1, hid)), full((1, 1)),
        ],
        out_specs=[
            pl.BlockSpec((t_len, hid, blk), lambda i: (0, 0, i)),
            pl.BlockSpec((1, blk), lambda i: (0, i)),
            pl.BlockSpec((1, blk), lambda i: (0, i)),
        ],
        out_shape=[
            jax.ShapeDtypeStruct((t_len, hid, npad), jnp.float32),
            jax.ShapeDtypeStruct((1, npad), jnp.float32),
            jax.ShapeDtypeStruct((1, npad), jnp.float32),
        ],
        scratch_shapes=[
            pltpu.VMEM((t_len, dk, blk), jnp.float32),
            pltpu.VMEM((t_len, dk, blk), jnp.float32),
            pltpu.VMEM((t_len, dk, blk), jnp.float32),
            pltpu.VMEM((t_len, heads, blk), jnp.float32),
        ],
        compiler_params=pltpu.CompilerParams(
            dimension_semantics=("arbitrary",)),
    )


def kernel(x, edge_index, edge_weight, Wp, bp, g0_Wl, g0_bl, g0_Wr, g0_br,
           g0_We, g0_att, g0_b, g1_Wl, g1_bl, g1_Wr, g1_br, g1_We, g1_att,
           g1_b, Wq, bq, Wk, bk, Wv, bv, Wo, bo, f_W1, f_b1, f_W2, f_b2,
           r_W1, r_b1, r_W2, r_b2):
    heads = g0_att.shape[0]
    src = edge_index[0]
    dst = edge_index[1]
    x_seq = x @ Wp + bp
    x_pool = x_seq.mean(axis=1)
    x_pool = _gat_layer(x_pool, src, dst, edge_weight,
                        g0_Wl, g0_bl, g0_Wr, g0_br, g0_We, g0_att, g0_b)
    x_pool = _gat_layer(x_pool, src, dst, edge_weight,
                        g1_Wl, g1_bl, g1_Wr, g1_br, g1_We, g1_att, g1_b)
    n, t_len, _ = x.shape
    hid = Wp.shape[1]
    dk = Wq.shape[1]
    blk = 512
    npad = -(-n // blk) * blk
    xsT = jnp.transpose(x_seq, (1, 2, 0))
    xsT = jnp.pad(xsT, ((0, 0), (0, 0), (0, npad - n)))
    xpT = jnp.pad(x_pool.T, ((0, 0), (0, npad - n)))
    tk = _make_temporal_kernel(npad, t_len, hid, dk, heads, blk)
    toutT, fout, rout = tk(
        xsT, xpT, Wq.T, bq[:, None], Wk.T, bk[:, None], Wv.T, bv[:, None],
        Wo.T, bo[:, None], f_W1.T, f_b1[:, None], f_W2.T, f_b2[:, None],
        r_W1.T, r_b1[:, None], r_W2.T, r_b2[:, None])
    temporal_out = jnp.transpose(toutT[:, :, :n], (0, 2, 1))
    forecast = fout[0, :n, None]
    risk = rout[0, :n, None]
    return (forecast, risk, temporal_out)


# speedup vs baseline: 87.5702x; 1.0385x over previous
import functools
import jax
import jax.numpy as jnp
from jax import lax
from jax.experimental import pallas as pl
from jax.experimental.pallas import tpu as pltpu
from jax.experimental.pallas import tpu_sc as plsc

NC = 2
NS = 16
L = 16
CH = 40


def _vgather(x, idx):
    dn = lax.GatherDimensionNumbers(
        offset_dims=(), collapsed_slice_dims=(0,), start_index_map=(0,))
    return lax.gather(x, idx[:, None], dn, (1,),
                      mode=lax.GatherScatterMode.PROMISE_IN_BOUNDS)


def _make_edge_kernel(n, e, hd2):
    nw = NC * NS
    assert e % (nw * CH) == 0
    ew_per = e // nw
    nch = ew_per // CH
    npairs = nch // 2
    assert nch % 2 == 0
    nvec = hd2 // L
    hv = 2
    nheads = nvec // hv
    wrows = CH
    assert n % wrows == 0
    nblk = n // wrows
    nblk_ceil = (nblk + NS - 1) // NS

    mesh = plsc.VectorSubcoreMesh(core_axis_name="c", subcore_axis_name="s")

    @functools.partial(
        pl.kernel,
        out_type=(jax.ShapeDtypeStruct((2 * n, hd2), jnp.float32),
                  jax.ShapeDtypeStruct((2 * n, L), jnp.float32)),
        mesh=mesh,
        compiler_params=pltpu.CompilerParams(use_tc_tiling_on_sc=False,
                                             needs_layout_passes=False),
        scratch_types=dict(
            acc_sh=pltpu.VMEM_SHARED((n, hd2), jnp.float32),
            accd_sh=pltpu.VMEM_SHARED((n, L), jnp.float32),
            ebuf0=pltpu.VMEM((4, CH), jnp.int32),
            ebuf1=pltpu.VMEM((4, CH), jnp.int32),
            sdst0=pltpu.VMEM((CH,), jnp.int32),
            sdst1=pltpu.VMEM((CH,), jnp.int32),
            xlr0=pltpu.VMEM((CH, hd2), jnp.float32),
            xlr1=pltpu.VMEM((CH, hd2), jnp.float32),
            xrr0=pltpu.VMEM((CH, hd2), jnp.float32),
            xrr1=pltpu.VMEM((CH, hd2), jnp.float32),
            stage0=pltpu.VMEM((CH, hd2), jnp.float32),
            stage1=pltpu.VMEM((CH, hd2), jnp.float32),
            staged0=pltpu.VMEM((CH, L), jnp.float32),
            staged1=pltpu.VMEM((CH, L), jnp.float32),
            wei=pltpu.VMEM((hd2,), jnp.float32),
            atti=pltpu.VMEM((hd2,), jnp.float32),
            gxl0=pltpu.SemaphoreType.DMA, gxl1=pltpu.SemaphoreType.DMA,
            gxr0=pltpu.SemaphoreType.DMA, gxr1=pltpu.SemaphoreType.DMA,
            isem0=pltpu.SemaphoreType.DMA, isem1=pltpu.SemaphoreType.DMA,
            ssem0=pltpu.SemaphoreType.DMA, ssem1=pltpu.SemaphoreType.DMA,
            ssn0=pltpu.SemaphoreType.DMA, ssn1=pltpu.SemaphoreType.DMA,
        ),
    )
    def edge_kernel(xl_hbm, xr_hbm, ep_hbm, we_hbm, att_hbm,
                    out_hbm, outd_hbm, acc_sh, accd_sh,
                    ebuf0, ebuf1, sdst0, sdst1, xlr0, xlr1, xrr0, xrr1,
                    stage0, stage1, staged0, staged1, wei, atti,
                    gxl0, gxl1, gxr0, gxr1, isem0, isem1,
                    ssem0, ssem1, ssn0, ssn1):
        cid = lax.axis_index("c")
        sid = lax.axis_index("s")
        wid = sid * NC + cid

        pltpu.sync_copy(we_hbm, wei)
        pltpu.sync_copy(att_hbm, atti)
        wv = [wei[pl.ds(i * L, L)] for i in range(nvec)]
        av = [atti[pl.ds(i * L, L)] for i in range(nvec)]
        lanes = lax.iota(jnp.int32, L)
        perms = [lanes ^ s for s in (8, 4, 2, 1)]
        hmask = [lanes == h for h in range(nheads)]

        zv = jnp.zeros((L,), jnp.float32)

        def zrow(r, _):
            for j in range(hd2 // L):
                stage0[r, pl.ds(j * L, L)] = zv
            staged0[r, pl.ds(0, L)] = zv
            return 0

        lax.fori_loop(0, CH, zrow, 0)

        def zblk(b, _):
            blk = b * NS + sid

            @pl.when(blk < nblk)
            def _():
                pltpu.sync_copy(stage0, acc_sh.at[pl.ds(blk * wrows, wrows)])
                pltpu.sync_copy(staged0,
                                accd_sh.at[pl.ds(blk * wrows, wrows)])

            return 0

        lax.fori_loop(0, nblk_ceil, zblk, 0)
        plsc.subcore_barrier()

        base0 = wid * ew_per
        par_refs = [
            (ebuf0, sdst0, xlr0, xrr0, stage0, staged0,
             gxl0, gxr0, isem0, ssem0, ssn0),
            (ebuf1, sdst1, xlr1, xrr1, stage1, staged1,
             gxl1, gxr1, isem1, ssem1, ssn1),
        ]

        for par in range(2):
            ebuf, _, xlr, xrr = par_refs[par][:4]
            gxl, gxr = par_refs[par][6:8]
            pltpu.sync_copy(ep_hbm.at[:, pl.ds(base0 + par * CH, CH)], ebuf)
            pltpu.async_copy(xl_hbm.at[ebuf.at[0]], xlr, gxl)
            pltpu.async_copy(xr_hbm.at[ebuf.at[1]], xrr, gxr)

        def compute_chunk(ebuf, xlr, xrr, stage, staged):
            def egroup(g, _):
                ewg = plsc.bitcast(ebuf[2, pl.ds(g * 8, L)], jnp.float32)
                for u0 in range(0, 8, 8):
                    ts = [g * 8 + u0 + i for i in range(8)]
                    qs = []
                    for i, t in enumerate(ts):
                        w = ewg[u0 + i]
                        qh = []
                        for h in range(nheads):
                            q = jnp.zeros((L,), jnp.float32)
                            for k in range(hv):
                                idx = h * hv + k
                                z = (xlr[t, pl.ds(idx * L, L)]
                                     + xrr[t, pl.ds(idx * L, L)]
                                     + w * wv[idx])
                                m = jnp.where(z >= 0.0, z, 0.2 * z)
                                q = q + m * av[idx]
                            qh.append(q)
                        qs.append(qh)
                    for p in perms:
                        qs = [[q + _vgather(q, p) for q in qh] for qh in qs]
                    for i, t in enumerate(ts):
                        den = jnp.zeros((L,), jnp.float32)
                        for h in range(nheads):
                            ex = jnp.exp(qs[i][h])
                            for k in range(hv):
                                idx = h * hv + k
                                stage[t, pl.ds(idx * L, L)] = (
                                    xlr[t, pl.ds(idx * L, L)] * ex)
                            den = jnp.where(hmask[h], ex, den)
                        staged[t, pl.ds(0, L)] = den
                return 0

            lax.fori_loop(0, CH // 8, egroup, 0)

        def pair(j, _):
            for par in range(2):
                (ebuf, sdst, xlr, xrr, stage, staged,
                 gxl, gxr, isem, ssem, ssn) = par_refs[par]
                c = 2 * j + par
                base = base0 + c * CH
                pltpu.make_async_copy(xl_hbm.at[ebuf.at[0]], xlr, gxl).wait()
                pltpu.make_async_copy(xr_hbm.at[ebuf.at[1]], xrr, gxr).wait()

                @pl.when(j >= 1)
                def _():
                    pltpu.make_async_copy(stage, acc_sh.at[sdst], ssem).wait()
                    pltpu.make_async_copy(staged, accd_sh.at[sdst],
                                          ssn).wait()

                pltpu.async_copy(ep_hbm.at[1, pl.ds(base, CH)], sdst, isem)

                compute_chunk(ebuf, xlr, xrr, stage, staged)

                @pl.when(j < npairs - 1)
                def _():
                    pltpu.sync_copy(
                        ep_hbm.at[:, pl.ds(base + 2 * CH, CH)], ebuf)
                    pltpu.async_copy(xl_hbm.at[ebuf.at[0]], xlr, gxl)
                    pltpu.async_copy(xr_hbm.at[ebuf.at[1]], xrr, gxr)

                pltpu.make_async_copy(ep_hbm.at[1, pl.ds(base, CH)], sdst,
                                      isem).wait()
                pltpu.async_copy(stage, acc_sh.at[sdst], ssem, add=True)
                pltpu.async_copy(staged, accd_sh.at[sdst], ssn, add=True)
            return 0

        lax.fori_loop(0, npairs, pair, 0)
        for par in range(2):
            (_, sdst, _, _, stage, staged,
             _, _, _, ssem, ssn) = par_refs[par]
            pltpu.make_async_copy(stage, acc_sh.at[sdst], ssem).wait()
            pltpu.make_async_copy(staged, accd_sh.at[sdst], ssn).wait()
        plsc.subcore_barrier()

        def wblk(b, _):
            blk = b * NS + sid

            @pl.when(blk < nblk)
            def _():
                r = blk * wrows
                pltpu.sync_copy(acc_sh.at[pl.ds(r, wrows)],
                                out_hbm.at[pl.ds(cid * n + r, wrows)])
                pltpu.sync_copy(accd_sh.at[pl.ds(r, wrows)],
                                outd_hbm.at[pl.ds(cid * n + r, wrows)])

            return 0

        lax.fori_loop(0, nblk_ceil, wblk, 0)

    return edge_kernel


def _gat_layer(xp, src, dst, ew, Wl, bl, Wr, br, We, att, b):
    n = xp.shape[0]
    e = src.shape[0]
    heads, c = att.shape
    hd2 = heads * c
    xl = xp @ Wl + bl
    xr = xp @ Wr + br
    epack = jnp.concatenate(
        [src[None], dst[None],
         jax.lax.bitcast_convert_type(ew, jnp.int32)[None],
         jnp.zeros((1, e), jnp.int32)], axis=0)
    wev = We.reshape(hd2)
    attv = att.reshape(hd2)
    ek = _make_edge_kernel(n, e, hd2)
    acc, accd = ek(xl, xr, epack, wev, attv)
    den = (accd[:n] + accd[n:])[:, :heads]
    num = (acc[:n] + acc[n:]).reshape(n, heads, c)
    out = (num / (den[:, :, None] + 1e-16)).mean(axis=1) + b
    return jax.nn.relu(out)


def _make_temporal_kernel(npad, t_len, hid, dk, heads, blk):
    nblk = npad // blk
    hd = dk // heads
    scale = hd ** -0.5

    def body(xsT, xpT, wqT, bq, wkT, bk, wvT, bv, woT, bo,
             fw1T, fb1, fw2T, fb2, rw1T, rb1, rw2T, rb2,
             toutT, fout, rout, q_s, k_s, v_s, srow_s):
        xp = xpT[...]

        def qkv(t, _):
            z = xsT[t] + xp
            q_s[t] = jnp.dot(wqT[...], z,
                             preferred_element_type=jnp.float32) + bq[...]
            k_s[t] = jnp.dot(wkT[...], z,
                             preferred_element_type=jnp.float32) + bk[...]
            v_s[t] = jnp.dot(wvT[...], z,
                             preferred_element_type=jnp.float32) + bv[...]
            return 0

        lax.fori_loop(0, t_len, qkv, 0)

        def attend(t, _):
            qt = q_s[t] * scale

            def score(u, _):
                prod = (qt * k_s[u]).reshape(heads, hd, blk)
                srow_s[u] = jnp.sum(prod, axis=1)
                return 0

            lax.fori_loop(0, t_len, score, 0)
            s = srow_s[...]
            m = jnp.max(s, axis=0, keepdims=True)
            e = jnp.exp(s - m)
            srow_s[...] = e / jnp.sum(e, axis=0, keepdims=True)

            def accum(u, c):
                return c + srow_s[u][:, None, :] * v_s[u].reshape(
                    heads, hd, blk)

            ctx = lax.fori_loop(
                0, t_len, accum, jnp.zeros((heads, hd, blk), jnp.float32))
            toutT[t] = (jnp.dot(woT[...], ctx.reshape(dk, blk),
                                preferred_element_type=jnp.float32) + bo[...])
            return 0

        lax.fori_loop(0, t_len, attend, 0)

        last = toutT[t_len - 1]
        h1 = jax.nn.relu(jnp.dot(fw1T[...], last,
                                 preferred_element_type=jnp.float32)
                         + fb1[...])
        fout[...] = jax.nn.relu(jnp.dot(fw2T[...], h1,
                                        preferred_element_type=jnp.float32)
                                + fb2[...])
        h2 = jax.nn.relu(jnp.dot(rw1T[...], last,
                                 preferred_element_type=jnp.float32)
                         + rb1[...])
        rout[...] = jax.nn.sigmoid(jnp.dot(rw2T[...], h2,
                                           preferred_element_type=jnp.float32)
                                   + rb2[...])

    full = lambda shape: pl.BlockSpec(shape, lambda i: (0,) * len(shape))
    return pl.pallas_call(
        body,
        grid=(nblk,),
        in_specs=[
            pl.BlockSpec((t_len, hid, blk), lambda i: (0, 0, i)),
            pl.BlockSpec((hid, blk), lambda i: (0, i)),
            full((dk, hid)), full((dk, 1)),
            full((dk, hid)), full((dk, 1)),
            full((dk, hid)), full((dk, 1)),
            full((hid, dk)), full((hid, 1)),
            full((hid, hid)), full((hid, 1)),
            full((1, hid)), full((1, 1)),
            full((hid, hid)), full((hid, 1)),
            full((1, hid)), full((1, 1)),
        ],
        out_specs=[
            pl.BlockSpec((t_len, hid, blk), lambda i: (0, 0, i)),
            pl.BlockSpec((1, blk), lambda i: (0, i)),
            pl.BlockSpec((1, blk), lambda i: (0, i)),
        ],
        out_shape=[
            jax.ShapeDtypeStruct((t_len, hid, npad), jnp.float32),
            jax.ShapeDtypeStruct((1, npad), jnp.float32),
            jax.ShapeDtypeStruct((1, npad), jnp.float32),
        ],
        scratch_shapes=[
            pltpu.VMEM((t_len, dk, blk), jnp.float32),
            pltpu.VMEM((t_len, dk, blk), jnp.float32),
            pltpu.VMEM((t_len, dk, blk), jnp.float32),
            pltpu.VMEM((t_len, heads, blk), jnp.float32),
        ],
        compiler_params=pltpu.CompilerParams(
            dimension_semantics=("arbitrary",)),
    )


def kernel(x, edge_index, edge_weight, Wp, bp, g0_Wl, g0_bl, g0_Wr, g0_br,
           g0_We, g0_att, g0_b, g1_Wl, g1_bl, g1_Wr, g1_br, g1_We, g1_att,
           g1_b, Wq, bq, Wk, bk, Wv, bv, Wo, bo, f_W1, f_b1, f_W2, f_b2,
           r_W1, r_b1, r_W2, r_b2):
    heads = g0_att.shape[0]
    src = edge_index[0]
    dst = edge_index[1]
    x_seq = x @ Wp + bp
    x_pool = x_seq.mean(axis=1)
    x_pool = _gat_layer(x_pool, src, dst, edge_weight,
                        g0_Wl, g0_bl, g0_Wr, g0_br, g0_We, g0_att, g0_b)
    x_pool = _gat_layer(x_pool, src, dst, edge_weight,
                        g1_Wl, g1_bl, g1_Wr, g1_br, g1_We, g1_att, g1_b)
    n, t_len, _ = x.shape
    hid = Wp.shape[1]
    dk = Wq.shape[1]
    blk = 512
    npad = -(-n // blk) * blk
    xsT = jnp.transpose(x_seq, (1, 2, 0))
    xsT = jnp.pad(xsT, ((0, 0), (0, 0), (0, npad - n)))
    xpT = jnp.pad(x_pool.T, ((0, 0), (0, npad - n)))
    tk = _make_temporal_kernel(npad, t_len, hid, dk, heads, blk)
    toutT, fout, rout = tk(
        xsT, xpT, Wq.T, bq[:, None], Wk.T, bk[:, None], Wv.T, bv[:, None],
        Wo.T, bo[:, None], f_W1.T, f_b1[:, None], f_W2.T, f_b2[:, None],
        r_W1.T, r_b1[:, None], r_W2.T, r_b2[:, None])
    temporal_out = jnp.transpose(toutT[:, :, :n], (0, 2, 1))
    forecast = fout[0, :n, None]
    risk = rout[0, :n, None]
    return (forecast, risk, temporal_out)
